# baseline (speedup 1.0000x reference)
import jax
import jax.numpy as jnp
from jax.experimental import pallas as pl

_F32 = jnp.float32
_BF16 = jnp.bfloat16


def _dot(a, b):
    return jnp.dot(a, b, preferred_element_type=_F32)


def _s2d(x):
    n, h, w, c = x.shape
    return (x.reshape(n, h // 2, 2, w // 2, 2, c)
             .transpose(0, 1, 3, 2, 4, 5)
             .reshape(n, h // 2, w // 2, 4 * c))


def _d2s(x, co):
    n, h, w, _ = x.shape
    return (x.reshape(n, h, w, 2, 2, co)
             .transpose(0, 1, 3, 2, 4, 5)
             .reshape(n, 2 * h, 2 * w, co))


def _enc_s2d_weights(w):
    o, c = w.shape[0], w.shape[1]
    ws = jnp.zeros((2, 2, 2, 2, c, o), w.dtype)
    for a, p, dy in ((0, 1, 0), (1, 0, 1), (1, 1, 2)):
        for b, q, dx in ((0, 1, 0), (1, 0, 1), (1, 1, 2)):
            ws = ws.at[a, b, p, q].set(w[:, :, dy, dx].T)
    return ws.reshape(2, 2, 4 * c, o)


_PHASE_TAP = {0: ((1, 0), (0, 1), (1, 1)),
              1: ((0, 1), (1, 1), (0, 2))}


def kernel(x, enc_w1, enc_b1, enc_w2, enc_b2, enc_w3, enc_b3, codebook,
           dec_w1, dec_b1, dec_w2, dec_b2, dec_w3, dec_b3):
    n, cin, hh, ww = x.shape
    h1, w1 = hh // 2, ww // 2
    h2, w2 = hh // 4, ww // 4
    c1 = enc_w1.shape[0]
    c2 = enc_w2.shape[0]
    ed = enc_w3.shape[0]
    ne = codebook.shape[0]
    d1 = dec_w1.shape[1]
    d2 = dec_w2.shape[1]

    f = _F32
    x_nhwc = x.transpose(0, 2, 3, 1).astype(_BF16)
    xs = _s2d(x_nhwc)
    xs4 = (x_nhwc.reshape(n, h2, 4, w2, 4, cin)
           .transpose(0, 1, 3, 2, 4, 5)
           .reshape(n, h2, w2, 16 * cin))
    xs4p = jnp.pad(xs4, ((0, 0), (1, 0), (1, 0), (0, 0)))

    _S4_MAP = {0: ((0, 3), (1, 0), (1, 1)), 1: ((1, 1), (1, 2), (1, 3))}
    w14 = jnp.zeros((2, 2, 16 * cin, 4 * c1), _F32)
    for pp in (0, 1):
        for qq in (0, 1):
            col0 = (pp * 2 + qq) * c1
            for dy in range(3):
                a, p4 = _S4_MAP[pp][dy]
                for dx in range(3):
                    b_, q4 = _S4_MAP[qq][dx]
                    row0 = (p4 * 4 + q4) * cin
                    w14 = w14.at[a, b_, row0:row0 + cin,
                                 col0:col0 + c1].add(enc_w1[:, :, dy, dx].T)
    w14 = w14.astype(_BF16)
    w2s = _enc_s2d_weights(enc_w2).astype(_BF16)
    w3r = enc_w3.reshape(ed, c2).T.astype(_BF16)
    cbt = codebook.T.astype(_BF16)
    wd1 = dec_w1.transpose(2, 3, 0, 1).astype(_BF16)
    wd2 = dec_w2.transpose(2, 3, 0, 1).astype(_BF16)
    wc = dec_w3.transpose(2, 3, 1, 0).astype(_BF16)
    b1 = enc_b1.reshape(1, -1)
    b2 = enc_b2.reshape(1, -1)
    b3 = enc_b3.reshape(1, -1)
    bd1 = dec_b1.reshape(1, -1)
    bd2 = dec_b2.reshape(1, -1)
    bc = dec_b3.reshape(1, -1)

    s1 = h2 // 4

    def k1_body(xs_ref, w_ref, b_ref, out_ref):
        bv = jnp.tile(b_ref[...], (1, 4))
        m = s1 * w2

        out_ref[0, 0:1, :, :] = jnp.zeros((1, w2 + 1, 4 * c1), _BF16)
        out_ref[0, :, 0:1, :] = jnp.zeros((h2 + 1, 1, 4 * c1), _BF16)

        def strip(s, carry):
            r0 = s * s1
            acc = jnp.zeros((m, 4 * c1), f)
            for a in (0, 1):
                for b_ in (0, 1):
                    v = xs_ref[0, pl.ds(r0 + a, s1), b_:b_ + w2,
                               :].reshape(m, 16 * cin)
                    acc = acc + _dot(v, w_ref[a, b_])
            out_ref[0, pl.ds(r0 + 1, s1), 1:w2 + 1, :] = (
                jnp.maximum(acc + bv, 0.0).astype(_BF16).reshape(s1, w2, 4 * c1))
            return carry

        jax.lax.fori_loop(0, h2 // s1, strip, 0)

    h1sp = pl.pallas_call(
        k1_body,
        grid=(n,),
        in_specs=[
            pl.BlockSpec((1, h2 + 1, w2 + 1, 16 * cin), lambda i: (i, 0, 0, 0)),
            pl.BlockSpec((2, 2, 16 * cin, 4 * c1), lambda i: (0, 0, 0, 0)),
            pl.BlockSpec((1, c1), lambda i: (0, 0)),
        ],
        out_specs=pl.BlockSpec((1, h2 + 1, w2 + 1, 4 * c1),
                               lambda i: (i, 0, 0, 0)),
        out_shape=jax.ShapeDtypeStruct((n, h2 + 1, w2 + 1, 4 * c1), _BF16),
    )(xs4p, w14, b1)


    def k2_body(hs_ref, w_ref, b_ref, w3_ref, b3_ref, out_ref):
        acc = jnp.zeros((h2 * w2, c2), f)
        for a in (0, 1):
            for b_ in (0, 1):
                v = hs_ref[0, a:a + h2, b_:b_ + w2, :].reshape(h2 * w2, 4 * c1)
                acc = acc + _dot(v, w_ref[a, b_])
        hval = jnp.maximum(acc + b_ref[...], 0.0).astype(_BF16)
        out_ref[0] = _dot(hval, w3_ref[...]) + b3_ref[...]

    z = pl.pallas_call(
        k2_body,
        grid=(n,),
        in_specs=[
            pl.BlockSpec((1, h2 + 1, w2 + 1, 4 * c1), lambda i: (i, 0, 0, 0)),
            pl.BlockSpec((2, 2, 4 * c1, c2), lambda i: (0, 0, 0, 0)),
            pl.BlockSpec((1, c2), lambda i: (0, 0)),
            pl.BlockSpec((c2, ed), lambda i: (0, 0)),
            pl.BlockSpec((1, ed), lambda i: (0, 0)),
        ],
        out_specs=pl.BlockSpec((1, h2 * w2, ed), lambda i: (i, 0, 0)),
        out_shape=jax.ShapeDtypeStruct((n, h2 * w2, ed), f),
    )(h1sp, w2s, b2, w3r, b3)

    m_tot = n * h2 * w2
    zf = z.reshape(m_tot, ed)
    vq_blk = 784 if m_tot % 784 == 0 else m_tot
    vq_grid = m_tot // vq_blk

    def k3_body(z_ref, cbt_ref, cb_ref, q_ref, sse_ref):
        zv = z_ref[...]
        scores = _dot(zv.astype(_BF16), cbt_ref[...])
        cf = cbt_ref[...].astype(f)
        cn = jnp.sum(cf * cf, axis=0, keepdims=True)
        d = cn - 2.0 * scores
        mn = jnp.min(d, axis=1, keepdims=True)
        iota = jax.lax.broadcasted_iota(jnp.int32, (1, ne), 1)
        idx = jnp.min(jnp.where(d <= mn, iota, ne), axis=1, keepdims=True)
        onehot = (iota == idx).astype(_BF16)
        qv = _dot(onehot, cb_ref[...])
        q_ref[...] = qv.astype(_BF16)
        diff = qv - zv
        part = jnp.sum(diff * diff)

        @pl.when(pl.program_id(0) == 0)
        def _():
            sse_ref[...] = jnp.zeros((1, 1), f)
        sse_ref[...] += jnp.full((1, 1), part, f)

    q, vq_sse = pl.pallas_call(
        k3_body,
        grid=(vq_grid,),
        in_specs=[
            pl.BlockSpec((vq_blk, ed), lambda i: (i, 0)),
            pl.BlockSpec((ed, ne), lambda i: (0, 0)),
            pl.BlockSpec((ne, ed), lambda i: (0, 0)),
        ],
        out_specs=[
            pl.BlockSpec((vq_blk, ed), lambda i: (i, 0)),
            pl.BlockSpec((1, 1), lambda i: (0, 0)),
        ],
        out_shape=[
            jax.ShapeDtypeStruct((m_tot, ed), _BF16),
            jax.ShapeDtypeStruct((1, 1), f),
        ],
    )(zf, cbt, codebook.astype(_BF16))

    def _tconv_call(inp, w, b, hin, ci, co, pad_out=False, wout=None):
        st = hin // 4
        if wout is None:
            wout = hin

        def body(q_ref, w_ref, b_ref, out_ref):
            bv = b_ref[...]
            m = st * hin

            if pad_out:
                out_ref[0, 0:1, :, :] = jnp.zeros((1, wout, 4 * co), _BF16)
                out_ref[0, hin + 1:hin + 2, :, :] = (
                    jnp.zeros((1, wout, 4 * co), _BF16))
                out_ref[0, :, 0:1, :] = jnp.zeros((hin + 2, 1, 4 * co), _BF16)
                out_ref[0, :, hin + 1:wout, :] = (
                    jnp.zeros((hin + 2, wout - hin - 1, 4 * co), _BF16))

            def strip(s, carry):
                r0 = s * st
                rows = q_ref[0, pl.ds(r0, st + 1), :, :]
                x00 = rows[0:st, 0:hin, :].reshape(m, ci)
                x01 = rows[0:st, 1:hin + 1, :].reshape(m, ci)
                ee = _dot(x00, w_ref[1, 1]) + bv
                eo = _dot(x01, w_ref[1, 0]) + _dot(x00, w_ref[1, 2]) + bv
                x10 = rows[1:st + 1, 0:hin, :].reshape(m, ci)
                oe = _dot(x10, w_ref[0, 1]) + _dot(x00, w_ref[2, 1]) + bv
                x11 = rows[1:st + 1, 1:hin + 1, :].reshape(m, ci)
                oo = (_dot(x11, w_ref[0, 0]) + _dot(x10, w_ref[0, 2])
                      + _dot(x01, w_ref[2, 0]) + _dot(x00, w_ref[2, 2]) + bv)
                out = jnp.concatenate([ee, eo, oe, oo], axis=1)
                val = jnp.maximum(out, 0.0).astype(_BF16).reshape(st, hin, 4 * co)
                if pad_out:
                    out_ref[0, pl.ds(r0 + 1, st), 1:hin + 1, :] = val
                else:
                    out_ref[0, pl.ds(r0, st), :, :] = val
                return carry

            jax.lax.fori_loop(0, hin // st, strip, 0)

        if pad_out:
            oshape = (n, hin + 2, wout, 4 * co)
            ospec = pl.BlockSpec((1, hin + 2, wout, 4 * co),
                                 lambda i: (i, 0, 0, 0))
        else:
            oshape = (n, hin, hin, 4 * co)
            ospec = pl.BlockSpec((1, hin, hin, 4 * co), lambda i: (i, 0, 0, 0))

        return pl.pallas_call(
            body,
            grid=(n,),
            in_specs=[
                pl.BlockSpec((1, hin + 1, hin + 1, ci), lambda i: (i, 0, 0, 0)),
                pl.BlockSpec((3, 3, ci, co), lambda i: (0, 0, 0, 0)),
                pl.BlockSpec((1, co), lambda i: (0, 0)),
            ],
            out_specs=ospec,
            out_shape=jax.ShapeDtypeStruct(oshape, _BF16),
        )(inp, w, b)

    qi = q.reshape(n, h2, w2, ed)
    qip = jnp.pad(qi, ((0, 0), (0, 1), (0, 1), (0, 0)))
    t1 = _tconv_call(qip, wd1, bd1, h2, ed, d1)

    t1i = _d2s(t1, d1)
    t1ip = jnp.pad(t1i, ((0, 0), (0, 1), (0, 1), (0, 0)))
    wout = 128
    t2 = _tconv_call(t1ip, wd2, bd2, h1, d1, d2,
                     pad_out=True, wout=wout)

    s6 = h1 // 4

    row_entries = {0: {}, 1: {}}
    for pp in (0, 1):
        for dy in range(3):
            p_, ry = _PHASE_TAP[pp][dy]
            row_entries[pp].setdefault(ry, []).append((p_, dy))

    shifts = [(ry, rx) for ry in range(3) for rx in range(3)]
    w9 = jnp.zeros((9, 4 * d2, 12), _F32)
    groups = []
    for k, (ry, rx) in enumerate(shifts):
        glist = []
        for pp in (0, 1):
            if ry not in row_entries[pp]:
                continue
            for qq in (0, 1):
                if rx not in row_entries[qq]:
                    continue
                g = len(glist)
                for p_, dy in row_entries[pp][ry]:
                    for q_, dx in row_entries[qq][rx]:
                        blk = (p_ * 2 + q_) * d2
                        w9 = w9.at[k, blk:blk + d2, g * 3:g * 3 + 3].add(
                            wc[dy, dx].astype(_F32))
                glist.append((g, (pp, qq)))
        groups.append(glist)
    w9 = w9.astype(_BF16)

    def k6_body(ph_ref, xs_ref, w9_ref, bc_ref, sse_ref):
        bcv = bc_ref[...]
        m = s6 * w1

        def strip(s, tot):
            r0 = s * s6
            accs = {}
            for pp in (0, 1):
                for qq in (0, 1):
                    xphase = xs_ref[0, pl.ds(r0, s6), :,
                                    (pp * 2 + qq) * cin:
                                    (pp * 2 + qq) * cin + cin].reshape(m, cin)
                    accs[(pp, qq)] = jnp.broadcast_to(bcv, (m, cin)) - xphase
            for k, (ry, rx) in enumerate(shifts):
                v = ph_ref[0, pl.ds(r0 + ry, s6), rx:rx + w1, :]
                y = _dot(v.reshape(m, 4 * d2), w9_ref[k])
                for g, pq in groups[k]:
                    accs[pq] = accs[pq] + y[:, g * 3:g * 3 + 3]
            for pq in accs:
                tot = tot + jnp.sum(accs[pq] * accs[pq])
            return tot

        total = jax.lax.fori_loop(0, h1 // s6, strip, jnp.zeros((), f))

        @pl.when(pl.program_id(0) == 0)
        def _():
            sse_ref[...] = jnp.zeros((1, 1), f)
        sse_ref[...] += jnp.full((1, 1), total, f)

    rec_sse = pl.pallas_call(
        k6_body,
        grid=(n,),
        in_specs=[
            pl.BlockSpec((1, h1 + 2, wout, 4 * d2), lambda i: (i, 0, 0, 0)),
            pl.BlockSpec((1, h1, w1, 4 * cin), lambda i: (i, 0, 0, 0)),
            pl.BlockSpec((9, 4 * d2, 12), lambda i: (0, 0, 0)),
            pl.BlockSpec((1, cin), lambda i: (0, 0)),
        ],
        out_specs=pl.BlockSpec((1, 1), lambda i: (0, 0)),
        out_shape=jax.ShapeDtypeStruct((1, 1), f),
    )(t2, xs, w9, bc)

    e_q_loss = 1.25 * vq_sse[0, 0] / (m_tot * ed)
    mse = rec_sse[0, 0] / (n * hh * ww * cin)
    recon_loss = mse / 1.0
    return (e_q_loss, recon_loss, mse)


# speedup vs baseline: 1.0199x; 1.0199x over previous
import jax
import jax.numpy as jnp
from jax.experimental import pallas as pl

_F32 = jnp.float32
_BF16 = jnp.bfloat16


def _dot(a, b):
    return jnp.dot(a, b, preferred_element_type=_F32)


def _s2d(x):
    n, h, w, c = x.shape
    return (x.reshape(n, h // 2, 2, w // 2, 2, c)
             .transpose(0, 1, 3, 2, 4, 5)
             .reshape(n, h // 2, w // 2, 4 * c))


def _d2s(x, co):
    n, h, w, _ = x.shape
    return (x.reshape(n, h, w, 2, 2, co)
             .transpose(0, 1, 3, 2, 4, 5)
             .reshape(n, 2 * h, 2 * w, co))


def _enc_s2d_weights(w):
    o, c = w.shape[0], w.shape[1]
    ws = jnp.zeros((2, 2, 2, 2, c, o), w.dtype)
    for a, p, dy in ((0, 1, 0), (1, 0, 1), (1, 1, 2)):
        for b, q, dx in ((0, 1, 0), (1, 0, 1), (1, 1, 2)):
            ws = ws.at[a, b, p, q].set(w[:, :, dy, dx].T)
    return ws.reshape(2, 2, 4 * c, o)


_PHASE_TAP = {0: ((1, 0), (0, 1), (1, 1)),
              1: ((0, 1), (1, 1), (0, 2))}


def kernel(x, enc_w1, enc_b1, enc_w2, enc_b2, enc_w3, enc_b3, codebook,
           dec_w1, dec_b1, dec_w2, dec_b2, dec_w3, dec_b3):
    n, cin, hh, ww = x.shape
    h1, w1 = hh // 2, ww // 2
    h2, w2 = hh // 4, ww // 4
    c1 = enc_w1.shape[0]
    c2 = enc_w2.shape[0]
    ed = enc_w3.shape[0]
    ne = codebook.shape[0]
    d1 = dec_w1.shape[1]
    d2 = dec_w2.shape[1]

    f = _F32
    x_nhwc = x.transpose(0, 2, 3, 1).astype(_BF16)
    xs = _s2d(x_nhwc)
    xs4 = (x_nhwc.reshape(n, h2, 4, w2, 4, cin)
           .transpose(0, 1, 3, 2, 4, 5)
           .reshape(n, h2, w2, 16 * cin))
    xs4p = jnp.pad(xs4, ((0, 0), (1, 0), (1, 0), (0, 0)))

    _S4_MAP = {0: ((0, 3), (1, 0), (1, 1)), 1: ((1, 1), (1, 2), (1, 3))}
    w14 = jnp.zeros((2, 2, 16 * cin, 4 * c1), _F32)
    for pp in (0, 1):
        for qq in (0, 1):
            col0 = (pp * 2 + qq) * c1
            for dy in range(3):
                a, p4 = _S4_MAP[pp][dy]
                for dx in range(3):
                    b_, q4 = _S4_MAP[qq][dx]
                    row0 = (p4 * 4 + q4) * cin
                    w14 = w14.at[a, b_, row0:row0 + cin,
                                 col0:col0 + c1].add(enc_w1[:, :, dy, dx].T)
    w14 = w14.astype(_BF16)
    w2s = _enc_s2d_weights(enc_w2).astype(_BF16)
    w3r = enc_w3.reshape(ed, c2).T.astype(_BF16)
    cbt = codebook.T.astype(_BF16)
    wd1 = dec_w1.transpose(2, 3, 0, 1).astype(_BF16)
    wd2 = dec_w2.transpose(2, 3, 0, 1).astype(_BF16)
    wc = dec_w3.transpose(2, 3, 1, 0).astype(_BF16)
    b1 = enc_b1.reshape(1, -1)
    b2 = enc_b2.reshape(1, -1)
    b3 = enc_b3.reshape(1, -1)
    bd1 = dec_b1.reshape(1, -1)
    bd2 = dec_b2.reshape(1, -1)
    bc = dec_b3.reshape(1, -1)

    s1 = h2 // 7

    def k1_body(xs_ref, w_ref, b_ref, out_ref):
        bv = jnp.tile(b_ref[...], (1, 4))
        m = s1 * w2

        out_ref[0, 0:1, :, :] = jnp.zeros((1, w2 + 1, 4 * c1), _BF16)
        out_ref[0, :, 0:1, :] = jnp.zeros((h2 + 1, 1, 4 * c1), _BF16)

        def strip(s, carry):
            r0 = s * s1
            acc = jnp.zeros((m, 4 * c1), f)
            for a in (0, 1):
                for b_ in (0, 1):
                    v = xs_ref[0, pl.ds(r0 + a, s1), b_:b_ + w2,
                               :].reshape(m, 16 * cin)
                    acc = acc + _dot(v, w_ref[a, b_])
            out_ref[0, pl.ds(r0 + 1, s1), 1:w2 + 1, :] = (
                jnp.maximum(acc + bv, 0.0).astype(_BF16).reshape(s1, w2, 4 * c1))
            return carry

        jax.lax.fori_loop(0, h2 // s1, strip, 0)

    h1sp = pl.pallas_call(
        k1_body,
        grid=(n,),
        in_specs=[
            pl.BlockSpec((1, h2 + 1, w2 + 1, 16 * cin), lambda i: (i, 0, 0, 0)),
            pl.BlockSpec((2, 2, 16 * cin, 4 * c1), lambda i: (0, 0, 0, 0)),
            pl.BlockSpec((1, c1), lambda i: (0, 0)),
        ],
        out_specs=pl.BlockSpec((1, h2 + 1, w2 + 1, 4 * c1),
                               lambda i: (i, 0, 0, 0)),
        out_shape=jax.ShapeDtypeStruct((n, h2 + 1, w2 + 1, 4 * c1), _BF16),
    )(xs4p, w14, b1)


    def k2_body(hs_ref, w_ref, b_ref, w3_ref, b3_ref, out_ref):
        acc = jnp.zeros((h2 * w2, c2), f)
        for a in (0, 1):
            for b_ in (0, 1):
                v = hs_ref[0, a:a + h2, b_:b_ + w2, :].reshape(h2 * w2, 4 * c1)
                acc = acc + _dot(v, w_ref[a, b_])
        hval = jnp.maximum(acc + b_ref[...], 0.0).astype(_BF16)
        out_ref[0] = _dot(hval, w3_ref[...]) + b3_ref[...]

    z = pl.pallas_call(
        k2_body,
        grid=(n,),
        in_specs=[
            pl.BlockSpec((1, h2 + 1, w2 + 1, 4 * c1), lambda i: (i, 0, 0, 0)),
            pl.BlockSpec((2, 2, 4 * c1, c2), lambda i: (0, 0, 0, 0)),
            pl.BlockSpec((1, c2), lambda i: (0, 0)),
            pl.BlockSpec((c2, ed), lambda i: (0, 0)),
            pl.BlockSpec((1, ed), lambda i: (0, 0)),
        ],
        out_specs=pl.BlockSpec((1, h2 * w2, ed), lambda i: (i, 0, 0)),
        out_shape=jax.ShapeDtypeStruct((n, h2 * w2, ed), f),
    )(h1sp, w2s, b2, w3r, b3)

    m_tot = n * h2 * w2
    zf = z.reshape(m_tot, ed)
    vq_blk = 784 if m_tot % 784 == 0 else m_tot
    vq_grid = m_tot // vq_blk

    def k3_body(z_ref, cbt_ref, cb_ref, q_ref, sse_ref):
        zv = z_ref[...]
        scores = _dot(zv.astype(_BF16), cbt_ref[...])
        cf = cbt_ref[...].astype(f)
        cn = jnp.sum(cf * cf, axis=0, keepdims=True)
        d = cn - 2.0 * scores
        mn = jnp.min(d, axis=1, keepdims=True)
        iota = jax.lax.broadcasted_iota(jnp.int32, (1, ne), 1)
        idx = jnp.min(jnp.where(d <= mn, iota, ne), axis=1, keepdims=True)
        onehot = (iota == idx).astype(_BF16)
        qv = _dot(onehot, cb_ref[...])
        q_ref[...] = qv.astype(_BF16)
        diff = qv - zv
        part = jnp.sum(diff * diff)

        @pl.when(pl.program_id(0) == 0)
        def _():
            sse_ref[...] = jnp.zeros((1, 1), f)
        sse_ref[...] += jnp.full((1, 1), part, f)

    q, vq_sse = pl.pallas_call(
        k3_body,
        grid=(vq_grid,),
        in_specs=[
            pl.BlockSpec((vq_blk, ed), lambda i: (i, 0)),
            pl.BlockSpec((ed, ne), lambda i: (0, 0)),
            pl.BlockSpec((ne, ed), lambda i: (0, 0)),
        ],
        out_specs=[
            pl.BlockSpec((vq_blk, ed), lambda i: (i, 0)),
            pl.BlockSpec((1, 1), lambda i: (0, 0)),
        ],
        out_shape=[
            jax.ShapeDtypeStruct((m_tot, ed), _BF16),
            jax.ShapeDtypeStruct((1, 1), f),
        ],
    )(zf, cbt, codebook.astype(_BF16))

    def _tconv_call(inp, w, b, hin, ci, co, pad_out=False, wout=None):
        st = hin // 7
        if wout is None:
            wout = hin

        def body(q_ref, w_ref, b_ref, out_ref):
            bv = b_ref[...]
            m = st * hin

            if pad_out:
                out_ref[0, 0:1, :, :] = jnp.zeros((1, wout, 4 * co), _BF16)
                out_ref[0, hin + 1:hin + 2, :, :] = (
                    jnp.zeros((1, wout, 4 * co), _BF16))
                out_ref[0, :, 0:1, :] = jnp.zeros((hin + 2, 1, 4 * co), _BF16)
                out_ref[0, :, hin + 1:wout, :] = (
                    jnp.zeros((hin + 2, wout - hin - 1, 4 * co), _BF16))

            def strip(s, carry):
                r0 = s * st
                rows = q_ref[0, pl.ds(r0, st + 1), :, :]
                x00 = rows[0:st, 0:hin, :].reshape(m, ci)
                x01 = rows[0:st, 1:hin + 1, :].reshape(m, ci)
                ee = _dot(x00, w_ref[1, 1]) + bv
                eo = _dot(x01, w_ref[1, 0]) + _dot(x00, w_ref[1, 2]) + bv
                x10 = rows[1:st + 1, 0:hin, :].reshape(m, ci)
                oe = _dot(x10, w_ref[0, 1]) + _dot(x00, w_ref[2, 1]) + bv
                x11 = rows[1:st + 1, 1:hin + 1, :].reshape(m, ci)
                oo = (_dot(x11, w_ref[0, 0]) + _dot(x10, w_ref[0, 2])
                      + _dot(x01, w_ref[2, 0]) + _dot(x00, w_ref[2, 2]) + bv)
                out = jnp.concatenate([ee, eo, oe, oo], axis=1)
                val = jnp.maximum(out, 0.0).astype(_BF16).reshape(st, hin, 4 * co)
                if pad_out:
                    out_ref[0, pl.ds(r0 + 1, st), 1:hin + 1, :] = val
                else:
                    out_ref[0, pl.ds(r0, st), :, :] = val
                return carry

            jax.lax.fori_loop(0, hin // st, strip, 0)

        if pad_out:
            oshape = (n, hin + 2, wout, 4 * co)
            ospec = pl.BlockSpec((1, hin + 2, wout, 4 * co),
                                 lambda i: (i, 0, 0, 0))
        else:
            oshape = (n, hin, hin, 4 * co)
            ospec = pl.BlockSpec((1, hin, hin, 4 * co), lambda i: (i, 0, 0, 0))

        return pl.pallas_call(
            body,
            grid=(n,),
            in_specs=[
                pl.BlockSpec((1, hin + 1, hin + 1, ci), lambda i: (i, 0, 0, 0)),
                pl.BlockSpec((3, 3, ci, co), lambda i: (0, 0, 0, 0)),
                pl.BlockSpec((1, co), lambda i: (0, 0)),
            ],
            out_specs=ospec,
            out_shape=jax.ShapeDtypeStruct(oshape, _BF16),
        )(inp, w, b)

    qi = q.reshape(n, h2, w2, ed)
    qip = jnp.pad(qi, ((0, 0), (0, 1), (0, 1), (0, 0)))
    t1 = _tconv_call(qip, wd1, bd1, h2, ed, d1)

    t1i = _d2s(t1, d1)
    t1ip = jnp.pad(t1i, ((0, 0), (0, 1), (0, 1), (0, 0)))
    wout = 128
    t2 = _tconv_call(t1ip, wd2, bd2, h1, d1, d2,
                     pad_out=True, wout=wout)

    s6 = h1 // 7

    row_entries = {0: {}, 1: {}}
    for pp in (0, 1):
        for dy in range(3):
            p_, ry = _PHASE_TAP[pp][dy]
            row_entries[pp].setdefault(ry, []).append((p_, dy))

    shifts = [(ry, rx) for ry in range(3) for rx in range(3)]
    w9 = jnp.zeros((9, 4 * d2, 12), _F32)
    groups = []
    for k, (ry, rx) in enumerate(shifts):
        glist = []
        for pp in (0, 1):
            if ry not in row_entries[pp]:
                continue
            for qq in (0, 1):
                if rx not in row_entries[qq]:
                    continue
                g = len(glist)
                for p_, dy in row_entries[pp][ry]:
                    for q_, dx in row_entries[qq][rx]:
                        blk = (p_ * 2 + q_) * d2
                        w9 = w9.at[k, blk:blk + d2, g * 3:g * 3 + 3].add(
                            wc[dy, dx].astype(_F32))
                glist.append((g, (pp, qq)))
        groups.append(glist)
    w9 = w9.astype(_BF16)

    def k6_body(ph_ref, xs_ref, w9_ref, bc_ref, sse_ref):
        bcv = bc_ref[...]
        m = s6 * w1

        def strip(s, tot):
            r0 = s * s6
            accs = {}
            for pp in (0, 1):
                for qq in (0, 1):
                    xphase = xs_ref[0, pl.ds(r0, s6), :,
                                    (pp * 2 + qq) * cin:
                                    (pp * 2 + qq) * cin + cin].reshape(m, cin)
                    accs[(pp, qq)] = jnp.broadcast_to(bcv, (m, cin)) - xphase
            for k, (ry, rx) in enumerate(shifts):
                v = ph_ref[0, pl.ds(r0 + ry, s6), rx:rx + w1, :]
                y = _dot(v.reshape(m, 4 * d2), w9_ref[k])
                for g, pq in groups[k]:
                    accs[pq] = accs[pq] + y[:, g * 3:g * 3 + 3]
            for pq in accs:
                tot = tot + jnp.sum(accs[pq] * accs[pq])
            return tot

        total = jax.lax.fori_loop(0, h1 // s6, strip, jnp.zeros((), f))

        @pl.when(pl.program_id(0) == 0)
        def _():
            sse_ref[...] = jnp.zeros((1, 1), f)
        sse_ref[...] += jnp.full((1, 1), total, f)

    rec_sse = pl.pallas_call(
        k6_body,
        grid=(n,),
        in_specs=[
            pl.BlockSpec((1, h1 + 2, wout, 4 * d2), lambda i: (i, 0, 0, 0)),
            pl.BlockSpec((1, h1, w1, 4 * cin), lambda i: (i, 0, 0, 0)),
            pl.BlockSpec((9, 4 * d2, 12), lambda i: (0, 0, 0)),
            pl.BlockSpec((1, cin), lambda i: (0, 0)),
        ],
        out_specs=pl.BlockSpec((1, 1), lambda i: (0, 0)),
        out_shape=jax.ShapeDtypeStruct((1, 1), f),
    )(t2, xs, w9, bc)

    e_q_loss = 1.25 * vq_sse[0, 0] / (m_tot * ed)
    mse = rec_sse[0, 0] / (n * hh * ww * cin)
    recon_loss = mse / 1.0
    return (e_q_loss, recon_loss, mse)
